# baseline (speedup 1.0000x reference)
import functools

import jax
import jax.numpy as jnp
from jax import lax
from jax.experimental import pallas as pl
from jax.experimental.pallas import tpu as pltpu
from jax.experimental.pallas import tpu_sc as plsc

_NP = 4096
_P = 64
_L = 16
_NC = 2
_NS = 16
_NW = _NC * _NS
_PPW = _NP // _NW
_CHUNK = 32

_BIG = 3.0e38


def _dir_min_sum(a_ref, b_ref, i, acc):
    ax = [a_ref[i, 0, pl.ds(c * _L, _L)] for c in range(4)]
    ay = [a_ref[i, 1, pl.ds(c * _L, _L)] for c in range(4)]
    az = [a_ref[i, 2, pl.ds(c * _L, _L)] for c in range(4)]

    def q_chunk_body(qc, mins):
        bx = b_ref[i, 0, pl.ds(qc * _L, _L)]
        by = b_ref[i, 1, pl.ds(qc * _L, _L)]
        bz = b_ref[i, 2, pl.ds(qc * _L, _L)]
        mins = list(mins)
        for l in range(_L):
            sx, sy, sz = bx[l], by[l], bz[l]
            for c in range(4):
                dx = ax[c] - sx
                dy = ay[c] - sy
                dz = az[c] - sz
                d2 = dx * dx + dy * dy + dz * dz
                mins[c] = jnp.minimum(mins[c], d2)
        return tuple(mins)

    init = tuple(jnp.full((_L,), _BIG, jnp.float32) for _ in range(4))
    mins = lax.fori_loop(0, _P // _L, q_chunk_body, init)
    return acc + mins[0] + mins[1] + mins[2] + mins[3]


def _sc_chamfer(pred_hbm, tgt_hbm, out_hbm, pred_v, tgt_v, acc_v):
    wid = lax.axis_index("s") * _NC + lax.axis_index("c")
    base = wid * _PPW

    def chunk_body(k, acc):
        pltpu.sync_copy(pred_hbm.at[pl.ds(base + k * _CHUNK, _CHUNK)], pred_v)
        pltpu.sync_copy(tgt_hbm.at[pl.ds(base + k * _CHUNK, _CHUNK)], tgt_v)

        def patch_body(i, acc):
            acc = _dir_min_sum(pred_v, tgt_v, i, acc)
            acc = _dir_min_sum(tgt_v, pred_v, i, acc)
            return acc

        return lax.fori_loop(0, _CHUNK, patch_body, acc)

    acc = lax.fori_loop(0, _PPW // _CHUNK, chunk_body,
                        jnp.zeros((_L,), jnp.float32))
    acc_v[...] = acc
    pltpu.sync_copy(acc_v, out_hbm.at[wid])


def kernel(pred_patches, target_patches):
    pred = pred_patches.reshape(_NP, _P, 3).swapaxes(1, 2)
    tgt = target_patches.reshape(_NP, _P, 3).swapaxes(1, 2)

    mesh = plsc.VectorSubcoreMesh(core_axis_name="c", subcore_axis_name="s")
    run = functools.partial(
        pl.kernel,
        mesh=mesh,
        out_type=jax.ShapeDtypeStruct((_NW, _L), jnp.float32),
        scratch_types=[
            pltpu.VMEM((_CHUNK, 3, _P), jnp.float32),
            pltpu.VMEM((_CHUNK, 3, _P), jnp.float32),
            pltpu.VMEM((_L,), jnp.float32),
        ],
    )(_sc_chamfer)
    partial_sums = run(pred, tgt)

    return jnp.sum(partial_sums) * (1.0 / (_NP * _P))


# speedup vs baseline: 8.6987x; 8.6987x over previous
import jax
import jax.numpy as jnp
from jax.experimental import pallas as pl

_NP = 4096
_P = 64
_BM = 256

_DN = (((1,), (1,)), ((0,), (0,)))


def _chamfer_body(pred_ref, tgt_ref, out_ref):
    @pl.when(pl.program_id(0) == 0)
    def _init():
        out_ref[...] = jnp.zeros_like(out_ref)

    p = pred_ref[...]
    t = tgt_ref[...]
    pn = jnp.sum(p * p, axis=1, keepdims=True)
    tn = jnp.sum(t * t, axis=1, keepdims=True)
    ones = jnp.ones_like(pn)

    lhs = jnp.concatenate([p, pn, ones], axis=1)
    rhs = jnp.concatenate([-2.0 * t, ones, tn], axis=1)

    d2 = jax.lax.dot_general(lhs, rhs, _DN, preferred_element_type=jnp.float32)

    fwd = jnp.min(d2, axis=2)
    bwd = jnp.min(d2, axis=1)
    step = jnp.sum(fwd) + jnp.sum(bwd)
    out_ref[...] += step.reshape(1, 1)


def kernel(pred_patches, target_patches):
    pred = pred_patches.reshape(_NP, _P, 3).swapaxes(1, 2)
    tgt = target_patches.reshape(_NP, _P, 3).swapaxes(1, 2)

    raw = pl.BlockSpec((_BM, 3, _P), lambda i: (i, 0, 0))
    total = pl.pallas_call(
        _chamfer_body,
        grid=(_NP // _BM,),
        in_specs=[raw, raw],
        out_specs=pl.BlockSpec((1, 1), lambda i: (0, 0)),
        out_shape=jax.ShapeDtypeStruct((1, 1), jnp.float32),
    )(pred, tgt)

    return total[0, 0] * (1.0 / (_NP * _P))
